# Initial kernel scaffold; baseline (speedup 1.0000x reference)
import jax
import jax.numpy as jnp
from jax.experimental import pallas as pl


def kernel(x):
    raise NotImplementedError("write your pallas kernel here")


# speedup vs baseline: 77.6224x; 77.6224x over previous
import functools

import jax
import jax.numpy as jnp
from jax import lax
from jax.experimental import pallas as pl
from jax.experimental.pallas import tpu as pltpu
from jax.experimental.pallas import tpu_sc as plsc

N, C, H, W = 8, 32, 512, 512
R = N * H
NW = 32
ROWS_PER_W = R // NW
L = 16
CHUNKS = W // L
NBUF = 2


def _onehot_body(x_hbm, out_hbm, xbuf, clsbuf, obuf,
                 in_sem0, in_sem1, out_sem0, out_sem1):
    cid = lax.axis_index("c")
    sid = lax.axis_index("s")
    wid = sid * 2 + cid
    row0 = wid * ROWS_PER_W

    iota = lax.iota(jnp.int32, L)
    ones_f = jnp.full((L,), 1.0, dtype=jnp.float32)
    zeros_f = jnp.zeros((L,), dtype=jnp.float32)
    zeros_i = jnp.zeros((L,), dtype=jnp.int32)

    in_sems = (in_sem0, in_sem1)
    out_sems = (out_sem0, out_sem1)

    def zero_row(c, carry):
        for b in range(NBUF):
            for j in range(CHUNKS):
                obuf[b, c, pl.ds(j * L, L)] = zeros_f
        return carry
    lax.fori_loop(0, C, zero_row, 0)
    for b in range(NBUF):
        for j in range(CHUNKS):
            clsbuf[b, pl.ds(j * L, L)] = zeros_i

    pltpu.async_copy(x_hbm.at[row0], xbuf.at[0], in_sem0)
    pltpu.async_copy(x_hbm.at[row0 + 1], xbuf.at[1], in_sem1)

    def outer(k, carry):
        for p in range(NBUF):
            r = row0 + k * NBUF + p
            n = lax.shift_right_logical(r, 9)
            h = lax.bitwise_and(r, H - 1)

            pltpu.make_async_copy(x_hbm.at[r], xbuf.at[p], in_sems[p]).wait()

            @pl.when(k > 0)
            def _wait_out():
                pltpu.make_async_copy(
                    obuf.at[p], out_hbm.at[n, :, h, :], out_sems[p]).wait()

            for j in range(CHUNKS):
                jv = iota + (j * L)
                cv = clsbuf[p, pl.ds(j * L, L)]
                plsc.store_scatter(obuf.at[p], [cv, jv], zeros_f)
            for j in range(CHUNKS):
                jv = iota + (j * L)
                xv = xbuf[p, pl.ds(j * L, L)]
                plsc.store_scatter(obuf.at[p], [xv, jv], ones_f)
                clsbuf[p, pl.ds(j * L, L)] = xv

            pltpu.async_copy(obuf.at[p], out_hbm.at[n, :, h, :], out_sems[p])

            @pl.when(k < ROWS_PER_W // NBUF - 1)
            def _prefetch():
                pltpu.async_copy(x_hbm.at[r + NBUF], xbuf.at[p], in_sems[p])
        return carry

    lax.fori_loop(0, ROWS_PER_W // NBUF, outer, 0)

    for p in range(NBUF):
        r = row0 + ROWS_PER_W - NBUF + p
        n = lax.shift_right_logical(r, 9)
        h = lax.bitwise_and(r, H - 1)
        pltpu.make_async_copy(
            obuf.at[p], out_hbm.at[n, :, h, :], out_sems[p]).wait()


@jax.jit
def _onehot_sc(x2):
    mesh = plsc.VectorSubcoreMesh(core_axis_name="c", subcore_axis_name="s")
    return pl.kernel(
        _onehot_body,
        mesh=mesh,
        compiler_params=pltpu.CompilerParams(needs_layout_passes=False),
        out_type=jax.ShapeDtypeStruct((N, C, H, W), jnp.float32),
        scratch_types=[
            pltpu.VMEM((NBUF, W), jnp.int32),
            pltpu.VMEM((NBUF, W), jnp.int32),
            pltpu.VMEM((NBUF, C, W), jnp.float32),
            pltpu.SemaphoreType.DMA,
            pltpu.SemaphoreType.DMA,
            pltpu.SemaphoreType.DMA,
            pltpu.SemaphoreType.DMA,
        ],
    )(x2)


def kernel(x):
    x2 = x.reshape(R, W).astype(jnp.int32)
    return _onehot_sc(x2)


# speedup vs baseline: 80.2202x; 1.0335x over previous
import functools

import jax
import jax.numpy as jnp
from jax import lax
from jax.experimental import pallas as pl
from jax.experimental.pallas import tpu as pltpu
from jax.experimental.pallas import tpu_sc as plsc

N, C, H, W = 8, 32, 512, 512
R = N * H
NW = 32
L = 16
HB = 8
WQ = 128
NJOBS_BLK = W // WQ
NBLK = R // HB
BLKS_PER_W = NBLK // NW
CHUNKS = HB * WQ // L
NBUF = 2


def _onehot_body(x_hbm, out_hbm, xbuf, clsbuf, obuf,
                 in_sem0, in_sem1, out_sem0, out_sem1):
    cid = lax.axis_index("c")
    sid = lax.axis_index("s")
    wid = sid * 2 + cid
    blk0 = wid * BLKS_PER_W

    iota = lax.iota(jnp.int32, L)
    ones_f = jnp.full((L,), 1.0, dtype=jnp.float32)
    zeros_f = jnp.zeros((L,), dtype=jnp.float32)
    zeros_i = jnp.zeros((L,), dtype=jnp.int32)

    in_sems = (in_sem0, in_sem1)
    out_sems = (out_sem0, out_sem1)

    def zero_row(c, carry):
        for b in range(NBUF):
            for hh in range(HB):
                for j in range(WQ // L):
                    obuf[b, c, hh, pl.ds(j * L, L)] = zeros_f
        return carry
    lax.fori_loop(0, C, zero_row, 0)
    for b in range(NBUF):
        for j in range(HB * WQ // L):
            clsbuf[b, pl.ds(j * L, L)] = zeros_i

    pltpu.async_copy(
        x_hbm.at[pl.ds(pl.multiple_of(blk0 * HB, HB), HB), :],
        xbuf.at[0], in_sem0)

    def outer(b, carry):
        blk = blk0 + b
        r = blk * HB
        n = lax.shift_right_logical(r, 9)
        h = pl.multiple_of(lax.bitwise_and(r, H - 1), HB)
        xb = lax.rem(b, 2)

        pltpu.make_async_copy(
            x_hbm.at[pl.ds(pl.multiple_of(blk * HB, HB), HB), :],
            xbuf.at[xb], in_sems[0]).wait()

        @pl.when(b < BLKS_PER_W - 1)
        def _prefetch():
            pltpu.async_copy(
                x_hbm.at[pl.ds(pl.multiple_of((blk + 1) * HB, HB), HB), :],
                xbuf.at[1 - xb], in_sems[0])

        for jj in range(NJOBS_BLK):
            w0 = jj * WQ
            p = jj % NBUF

            @pl.when(jnp.logical_or(b > 0, jj >= NBUF))
            def _wait_out():
                pltpu.make_async_copy(
                    obuf.at[p],
                    out_hbm.at[n, :, pl.ds(h, HB), pl.ds(w0, WQ)],
                    out_sems[p]).wait()

            for ch in range(CHUNKS):
                hh = ch // (WQ // L)
                wl = (ch % (WQ // L)) * L
                jv = iota + wl
                hv = jnp.full((L,), hh, dtype=jnp.int32)
                cv = clsbuf[p, pl.ds(ch * L, L)]
                plsc.store_scatter(obuf.at[p], [cv, hv, jv], zeros_f)
            for ch in range(CHUNKS):
                hh = ch // (WQ // L)
                wl = (ch % (WQ // L)) * L
                jv = iota + wl
                hv = jnp.full((L,), hh, dtype=jnp.int32)
                xv = xbuf[xb, hh, pl.ds(w0 + wl, L)]
                plsc.store_scatter(obuf.at[p], [xv, hv, jv], ones_f)
                clsbuf[p, pl.ds(ch * L, L)] = xv

            pltpu.async_copy(
                obuf.at[p],
                out_hbm.at[n, :, pl.ds(h, HB), pl.ds(w0, WQ)],
                out_sems[p])
        return carry

    lax.fori_loop(0, BLKS_PER_W, outer, 0)

    blk = blk0 + BLKS_PER_W - 1
    r = blk * HB
    n = lax.shift_right_logical(r, 9)
    h = pl.multiple_of(lax.bitwise_and(r, H - 1), HB)
    for jj in range(NJOBS_BLK - NBUF, NJOBS_BLK):
        w0 = jj * WQ
        p = jj % NBUF
        pltpu.make_async_copy(
            obuf.at[p],
            out_hbm.at[n, :, pl.ds(h, HB), pl.ds(w0, WQ)],
            out_sems[p]).wait()


@jax.jit
def _onehot_sc(x2):
    mesh = plsc.VectorSubcoreMesh(core_axis_name="c", subcore_axis_name="s")
    return pl.kernel(
        _onehot_body,
        mesh=mesh,
        compiler_params=pltpu.CompilerParams(needs_layout_passes=False),
        out_type=jax.ShapeDtypeStruct((N, C, H, W), jnp.float32),
        scratch_types=[
            pltpu.VMEM((NBUF, HB, W), jnp.int32),
            pltpu.VMEM((NBUF, HB * WQ), jnp.int32),
            pltpu.VMEM((NBUF, C, HB, WQ), jnp.float32),
            pltpu.SemaphoreType.DMA,
            pltpu.SemaphoreType.DMA,
            pltpu.SemaphoreType.DMA,
            pltpu.SemaphoreType.DMA,
        ],
    )(x2)


def kernel(x):
    x2 = x.reshape(R, W).astype(jnp.int32)
    return _onehot_sc(x2)


# speedup vs baseline: 80.2746x; 1.0007x over previous
import functools

import jax
import jax.numpy as jnp
from jax import lax
from jax.experimental import pallas as pl
from jax.experimental.pallas import tpu as pltpu
from jax.experimental.pallas import tpu_sc as plsc

N, C, H, W = 8, 32, 512, 512
R = N * H
NW = 32
L = 16
HB = 8
WQ = 128
NJOBS_BLK = W // WQ
NBLK = R // HB
BLKS_PER_W = NBLK // NW
CHUNKS = HB * WQ // L
NBUF = 2


def _onehot_body(x_hbm, out_hbm, xbuf, clsbuf, obuf,
                 in_sem0, in_sem1, out_sem0, out_sem1):
    cid = lax.axis_index("c")
    sid = lax.axis_index("s")
    wid = sid * 2 + cid
    blk0 = wid * BLKS_PER_W

    iota = lax.iota(jnp.int32, L)
    ones_f = jnp.full((L,), 1.0, dtype=jnp.float32)
    zeros_f = jnp.zeros((L,), dtype=jnp.float32)
    zeros_i = jnp.zeros((L,), dtype=jnp.int32)

    in_sems = (in_sem0, in_sem1)
    out_sems = (out_sem0, out_sem1)

    pltpu.async_copy(
        x_hbm.at[pl.ds(pl.multiple_of(blk0 * HB, HB), HB), :],
        xbuf.at[0], in_sem0)

    def zero_row(c, carry):
        for b in range(NBUF):
            for hh in range(HB):
                for j in range(WQ // L):
                    obuf[b, c, hh, pl.ds(j * L, L)] = zeros_f
        return carry
    lax.fori_loop(0, C, zero_row, 0)
    for b in range(NBUF):
        for j in range(HB * WQ // L):
            clsbuf[b, pl.ds(j * L, L)] = zeros_i

    def outer(b, carry):
        blk = blk0 + b
        r = blk * HB
        n = lax.shift_right_logical(r, 9)
        h = pl.multiple_of(lax.bitwise_and(r, H - 1), HB)
        xb = lax.rem(b, 2)

        @pl.when(xb == 0)
        def _wait_in0():
            pltpu.make_async_copy(
                x_hbm.at[pl.ds(pl.multiple_of(blk * HB, HB), HB), :],
                xbuf.at[0], in_sems[0]).wait()

        @pl.when(xb == 1)
        def _wait_in1():
            pltpu.make_async_copy(
                x_hbm.at[pl.ds(pl.multiple_of(blk * HB, HB), HB), :],
                xbuf.at[1], in_sems[1]).wait()

        @pl.when(jnp.logical_and(b < BLKS_PER_W - 1, xb == 0))
        def _prefetch1():
            pltpu.async_copy(
                x_hbm.at[pl.ds(pl.multiple_of((blk + 1) * HB, HB), HB), :],
                xbuf.at[1], in_sems[1])

        @pl.when(jnp.logical_and(b < BLKS_PER_W - 1, xb == 1))
        def _prefetch0():
            pltpu.async_copy(
                x_hbm.at[pl.ds(pl.multiple_of((blk + 1) * HB, HB), HB), :],
                xbuf.at[0], in_sems[0])

        for jj in range(NJOBS_BLK):
            w0 = jj * WQ
            p = jj % NBUF

            @pl.when(jnp.logical_or(b > 0, jj >= NBUF))
            def _wait_out():
                pltpu.make_async_copy(
                    obuf.at[p],
                    out_hbm.at[n, :, pl.ds(h, HB), pl.ds(w0, WQ)],
                    out_sems[p]).wait()

            for ch in range(CHUNKS):
                hh = ch // (WQ // L)
                wl = (ch % (WQ // L)) * L
                jv = iota + wl
                hv = jnp.full((L,), hh, dtype=jnp.int32)
                cv = clsbuf[p, pl.ds(ch * L, L)]
                plsc.store_scatter(obuf.at[p], [cv, hv, jv], zeros_f)
            for ch in range(CHUNKS):
                hh = ch // (WQ // L)
                wl = (ch % (WQ // L)) * L
                jv = iota + wl
                hv = jnp.full((L,), hh, dtype=jnp.int32)
                xv = xbuf[xb, hh, pl.ds(w0 + wl, L)]
                plsc.store_scatter(obuf.at[p], [xv, hv, jv], ones_f)
                clsbuf[p, pl.ds(ch * L, L)] = xv

            pltpu.async_copy(
                obuf.at[p],
                out_hbm.at[n, :, pl.ds(h, HB), pl.ds(w0, WQ)],
                out_sems[p])
        return carry

    lax.fori_loop(0, BLKS_PER_W, outer, 0)

    blk = blk0 + BLKS_PER_W - 1
    r = blk * HB
    n = lax.shift_right_logical(r, 9)
    h = pl.multiple_of(lax.bitwise_and(r, H - 1), HB)
    for jj in range(NJOBS_BLK - NBUF, NJOBS_BLK):
        w0 = jj * WQ
        p = jj % NBUF
        pltpu.make_async_copy(
            obuf.at[p],
            out_hbm.at[n, :, pl.ds(h, HB), pl.ds(w0, WQ)],
            out_sems[p]).wait()


@jax.jit
def _onehot_sc(x2):
    mesh = plsc.VectorSubcoreMesh(core_axis_name="c", subcore_axis_name="s")
    return pl.kernel(
        _onehot_body,
        mesh=mesh,
        compiler_params=pltpu.CompilerParams(needs_layout_passes=False),
        out_type=jax.ShapeDtypeStruct((N, C, H, W), jnp.float32),
        scratch_types=[
            pltpu.VMEM((NBUF, HB, W), jnp.int32),
            pltpu.VMEM((NBUF, HB * WQ), jnp.int32),
            pltpu.VMEM((NBUF, C, HB, WQ), jnp.float32),
            pltpu.SemaphoreType.DMA,
            pltpu.SemaphoreType.DMA,
            pltpu.SemaphoreType.DMA,
            pltpu.SemaphoreType.DMA,
        ],
    )(x2)


def kernel(x):
    x2 = x.reshape(R, W).astype(jnp.int32)
    return _onehot_sc(x2)
